# baseline (speedup 1.0000x reference)
import functools

import jax
import jax.numpy as jnp
from jax import lax
from jax.experimental import pallas as pl
from jax.experimental.pallas import tpu as pltpu
from jax.experimental.pallas import tpu_sc as plsc

T = 4096
H = 1024
I = 512
E = 8
TOP_K = 2

BT = 256
P = T * TOP_K + E * BT
NB = P // BT
NBP = 48
TK = T * TOP_K

NLANE = 16
NW_PLAN = 16
NW = 32

BT_R = 512


def _router_body(x_ref, gw_ref, bias_ref, meta_ref):
    x = x_ref[...]
    logits = lax.dot_general(x, gw_ref[...], (((1,), (1,)), ((), ())),
                             preferred_element_type=jnp.float32)
    s = jax.nn.sigmoid(logits)
    sc = s + bias_ref[...]
    ii = lax.broadcasted_iota(jnp.int32, (BT_R, E), 1)
    m1 = jnp.max(sc, axis=1, keepdims=True)
    i1 = jnp.min(jnp.where(sc >= m1, ii, E), axis=1, keepdims=True)
    sc2 = jnp.where(ii == i1, -jnp.inf, sc)
    m2 = jnp.max(sc2, axis=1, keepdims=True)
    i2 = jnp.min(jnp.where(sc2 >= m2, ii, E), axis=1, keepdims=True)
    s1 = jnp.sum(jnp.where(ii == i1, s, 0.0), axis=1, keepdims=True)
    s2 = jnp.sum(jnp.where(ii == i2, s, 0.0), axis=1, keepdims=True)
    denom = s1 + s2
    meta_ref[...] = jnp.concatenate(
        [i1.astype(jnp.float32), i2.astype(jnp.float32),
         s1 / denom, s2 / denom,
         jnp.zeros((BT_R, 4), jnp.float32)], axis=1)


def _router(x, gw, bias):
    return pl.pallas_call(
        _router_body,
        grid=(T // BT_R,),
        in_specs=[
            pl.BlockSpec((BT_R, H), lambda t: (t, 0)),
            pl.BlockSpec((E, H), lambda t: (0, 0)),
            pl.BlockSpec((1, E), lambda t: (0, 0)),
        ],
        out_specs=pl.BlockSpec((BT_R, 8), lambda t: (t, 0)),
        out_shape=jax.ShapeDtypeStruct((T, 8), jnp.float32),
        compiler_params=pltpu.CompilerParams(
            dimension_semantics=("arbitrary",)),
    )(x, gw, bias)


def _shared_body(x_ref, sgu_ref, sd_ref, out_ref):
    xb = x_ref[...].astype(jnp.bfloat16)
    gu = jnp.dot(xb, sgu_ref[...], preferred_element_type=jnp.float32)
    h = (jax.nn.silu(gu[:, :I]) * gu[:, I:]).astype(jnp.bfloat16)
    out_ref[...] = jnp.dot(h, sd_ref[...], preferred_element_type=jnp.float32)


def _shared(x, sgu16, sd16):
    return pl.pallas_call(
        _shared_body,
        grid=(T // BT_R,),
        in_specs=[
            pl.BlockSpec((BT_R, H), lambda t: (t, 0)),
            pl.BlockSpec((H, 2 * I), lambda t: (0, 0)),
            pl.BlockSpec((I, H), lambda t: (0, 0)),
        ],
        out_specs=pl.BlockSpec((BT_R, H), lambda t: (t, 0)),
        out_shape=jax.ShapeDtypeStruct((T, H), jnp.float32),
        compiler_params=pltpu.CompilerParams(
            dimension_semantics=("arbitrary",)),
    )(x, sgu16, sd16)


_PAIRS_W = TK // NW_PLAN
_ZERO_W = P // NW_PLAN
_META_W = 8 * (T // NW_PLAN)


def _plan_body(meta_hbm, tok_hbm, wts_hbm, blk_hbm, pos_hbm, hstage_hbm,
               mv, slot_v, tok_v, wt_v, zvi, zvf, hist_v, allh_v, blk_v,
               sem):
    wid = lax.axis_index("s")
    lane = lax.broadcasted_iota(jnp.int32, (NLANE,), 0)
    zeros16i = jnp.zeros((NLANE,), jnp.int32)

    for i in range(_ZERO_W // NLANE):
        zvi[pl.ds(NLANE * i, NLANE)] = zeros16i
        zvf[pl.ds(NLANE * i, NLANE)] = jnp.zeros((NLANE,), jnp.float32)
    pltpu.sync_copy(zvi, tok_hbm.at[pl.ds(wid * _ZERO_W, _ZERO_W)])
    pltpu.sync_copy(zvf, wts_hbm.at[pl.ds(wid * _ZERO_W, _ZERO_W)])

    pltpu.sync_copy(meta_hbm.at[pl.ds(wid * _META_W, _META_W)], mv)
    h = zeros16i
    for j in range(_PAIRS_W // NLANE):
        pi = j * NLANE + lane
        idx_e = ((pi >> 1) << 3) | (pi & 1)
        v = plsc.load_gather(mv, [idx_e]).astype(jnp.int32)
        for e in range(E):
            cnt = jnp.sum(jnp.where(v == e, 1, 0))
            h = h + jnp.where(lane == e, cnt, 0)
    hist_v[...] = h
    pltpu.sync_copy(hist_v, hstage_hbm.at[pl.ds(wid * NLANE, NLANE)])
    plsc.subcore_barrier()

    pltpu.sync_copy(hstage_hbm, allh_v)
    c_tot = zeros16i
    prior = zeros16i
    for w in range(NW_PLAN):
        row = allh_v[pl.ds(w * NLANE, NLANE)]
        c_tot = c_tot + row
        prior = prior + jnp.where(jnp.broadcast_to(w < wid, (NLANE,)), row, 0)
    r = ((c_tot + (BT - 1)) >> 8) << 8
    start = plsc.cumsum(r) - r
    b_run = start + prior

    for j in range(_PAIRS_W // NLANE):
        pi = j * NLANE + lane
        idx_e = ((pi >> 1) << 3) | (pi & 1)
        v = plsc.load_gather(mv, [idx_e]).astype(jnp.int32)
        w_f = plsc.load_gather(mv, [idx_e + 2])
        slot = zeros16i
        for e in range(E):
            ind = jnp.where(v == e, 1, 0)
            prefix = plsc.cumsum(ind)
            cnt = jnp.sum(ind)
            base_e = jnp.sum(jnp.where(lane == e, b_run, 0))
            slot = jnp.where(v == e, base_e + prefix - 1, slot)
            b_run = b_run + jnp.where(lane == e, cnt, 0)
        sl = pl.ds(j * NLANE, NLANE)
        slot_v[sl] = slot
        tok_v[sl] = wid * (_PAIRS_W // 2) + (pi >> 1)
        wt_v[sl] = w_f
    pltpu.async_copy(tok_v, tok_hbm.at[slot_v], sem).wait()
    pltpu.async_copy(wt_v, wts_hbm.at[slot_v], sem).wait()
    pltpu.sync_copy(slot_v, pos_hbm.at[pl.ds(wid * _PAIRS_W, _PAIRS_W)])

    @pl.when(wid == 0)
    def _():
        nb = (c_tot + (BT - 1)) >> 8
        sb = start >> 8
        for j in range(NBP // NLANE):
            bi = j * NLANE + lane
            acc = zeros16i
            for e in range(1, E):
                sb_e = jnp.sum(jnp.where(lane == e, sb, 0))
                nb_e = jnp.sum(jnp.where(lane == e, nb, 0))
                m = (bi >= sb_e) & (bi < sb_e + nb_e)
                acc = acc + jnp.where(m, e, 0)
            blk_v[pl.ds(j * NLANE, NLANE)] = acc
        pltpu.sync_copy(blk_v, blk_hbm)


def _plan(meta_flat):
    f = pl.kernel(
        _plan_body,
        out_type=[
            jax.ShapeDtypeStruct((P,), jnp.int32),
            jax.ShapeDtypeStruct((P,), jnp.float32),
            jax.ShapeDtypeStruct((NBP,), jnp.int32),
            jax.ShapeDtypeStruct((TK,), jnp.int32),
            jax.ShapeDtypeStruct((NW_PLAN * NLANE,), jnp.int32),
        ],
        mesh=plsc.VectorSubcoreMesh(core_axis_name="c", subcore_axis_name="s",
                                    num_cores=1),
        scratch_types=[
            pltpu.VMEM((_META_W,), jnp.float32),
            pltpu.VMEM((_PAIRS_W,), jnp.int32),
            pltpu.VMEM((_PAIRS_W,), jnp.int32),
            pltpu.VMEM((_PAIRS_W,), jnp.float32),
            pltpu.VMEM((_ZERO_W,), jnp.int32),
            pltpu.VMEM((_ZERO_W,), jnp.float32),
            pltpu.VMEM((NLANE,), jnp.int32),
            pltpu.VMEM((NW_PLAN * NLANE,), jnp.int32),
            pltpu.VMEM((NBP,), jnp.int32),
            pltpu.SemaphoreType.DMA,
        ],
        compiler_params=pltpu.CompilerParams(needs_layout_passes=False),
    )
    return f(meta_flat)


_ROWS_W = P // NW
_GCH = 64


def _gather_body(tok_hbm, x_hbm, xg_hbm, idx_v, row_v, sem):
    wid = lax.axis_index("s") * 2 + lax.axis_index("c")
    base = wid * _ROWS_W
    for c in range(_ROWS_W // _GCH):
        off = base + c * _GCH
        pltpu.sync_copy(tok_hbm.at[pl.ds(off, _GCH)], idx_v)
        pltpu.async_copy(x_hbm.at[idx_v], row_v, sem).wait()
        pltpu.sync_copy(row_v, xg_hbm.at[pl.ds(off, _GCH)])


def _gather(tok, x):
    f = pl.kernel(
        _gather_body,
        out_type=[jax.ShapeDtypeStruct((P, H), jnp.float32)],
        mesh=plsc.VectorSubcoreMesh(core_axis_name="c", subcore_axis_name="s"),
        scratch_types=[
            pltpu.VMEM((_GCH,), jnp.int32),
            pltpu.VMEM((_GCH, H), jnp.float32),
            pltpu.SemaphoreType.DMA,
        ],
        compiler_params=pltpu.CompilerParams(needs_layout_passes=False),
    )
    return f(tok, x)


def _ffn_body(be_ref, xg_ref, wgu_ref, wd_ref, wt_ref, out_ref):
    xb = xg_ref[...].astype(jnp.bfloat16)
    gu = jnp.dot(xb, wgu_ref[0], preferred_element_type=jnp.float32)
    h = (jax.nn.silu(gu[:, :I]) * gu[:, I:]).astype(jnp.bfloat16)
    oe = jnp.dot(h, wd_ref[0], preferred_element_type=jnp.float32)
    out_ref[...] = oe * wt_ref[0, 0, :][:, None]


def _ffn(blk, xg, wgu16, wd16, wt3):
    grid_spec = pltpu.PrefetchScalarGridSpec(
        num_scalar_prefetch=1,
        grid=(NB,),
        in_specs=[
            pl.BlockSpec((BT, H), lambda b, be: (b, 0)),
            pl.BlockSpec((1, H, 2 * I), lambda b, be: (be[b], 0, 0)),
            pl.BlockSpec((1, I, H), lambda b, be: (be[b], 0, 0)),
            pl.BlockSpec((1, 1, BT), lambda b, be: (b, 0, 0)),
        ],
        out_specs=pl.BlockSpec((BT, H), lambda b, be: (b, 0)),
    )
    return pl.pallas_call(
        _ffn_body,
        grid_spec=grid_spec,
        out_shape=jax.ShapeDtypeStruct((P, H), jnp.float32),
        compiler_params=pltpu.CompilerParams(
            dimension_semantics=("arbitrary",)),
    )(blk, xg, wgu16, wd16, wt3)


_TOK_W = T // NW
_CCH = 32


def _combine_body(pos_hbm, routed_hbm, sh_hbm, out_hbm, idx_v, g_v, acc_v, sem):
    wid = lax.axis_index("s") * 2 + lax.axis_index("c")
    tb0 = wid * _TOK_W
    for c in range(_TOK_W // _CCH):
        tb = tb0 + c * _CCH
        pltpu.sync_copy(pos_hbm.at[pl.ds(2 * tb, 2 * _CCH)], idx_v)
        cp = pltpu.async_copy(routed_hbm.at[idx_v], g_v, sem)
        pltpu.sync_copy(sh_hbm.at[pl.ds(tb, _CCH)], acc_v)
        cp.wait()

        def body(i, carry):
            for jj in range(H // NLANE):
                sl = pl.ds(NLANE * jj, NLANE)
                acc_v[i, sl] += g_v[2 * i, sl] + g_v[2 * i + 1, sl]
            return carry

        lax.fori_loop(0, _CCH, body, 0)
        pltpu.sync_copy(acc_v, out_hbm.at[pl.ds(tb, _CCH)])


def _combine(pos, routed, shared_out):
    f = pl.kernel(
        _combine_body,
        out_type=[jax.ShapeDtypeStruct((T, H), jnp.float32)],
        mesh=plsc.VectorSubcoreMesh(core_axis_name="c", subcore_axis_name="s"),
        scratch_types=[
            pltpu.VMEM((2 * _CCH,), jnp.int32),
            pltpu.VMEM((2 * _CCH, H), jnp.float32),
            pltpu.VMEM((_CCH, H), jnp.float32),
            pltpu.SemaphoreType.DMA,
        ],
        compiler_params=pltpu.CompilerParams(needs_layout_passes=False),
    )
    return f(pos, routed, shared_out)


def kernel(hidden_states, gate_weight, correction_bias, w_gate_up, w_down,
           shared_gate_up, shared_down):
    bf = jnp.bfloat16
    x = hidden_states
    meta = _router(x, gate_weight, correction_bias)
    shared_out = _shared(x, shared_gate_up.astype(bf), shared_down.astype(bf))
    tok, wt, blk, pos, _ = _plan(meta.reshape(-1))
    (xg,) = _gather(tok, x)
    routed = _ffn(blk, xg, w_gate_up.astype(bf), w_down.astype(bf),
                  wt.reshape(NB, 1, BT))
    (out,) = _combine(pos, routed, shared_out)
    return out


# speedup vs baseline: 1.0599x; 1.0599x over previous
import functools

import jax
import jax.numpy as jnp
from jax import lax
from jax.experimental import pallas as pl
from jax.experimental.pallas import tpu as pltpu
from jax.experimental.pallas import tpu_sc as plsc

T = 4096
H = 1024
I = 512
E = 8
TOP_K = 2

BT = 256
P = T * TOP_K + E * BT
NB = P // BT
NBP = 48
TK = T * TOP_K

NLANE = 16
NW_PLAN = 16
NW = 32

BT_R = 512


def _router_body(x_ref, gw_ref, bias_ref, meta_ref):
    x = x_ref[...]
    logits = lax.dot_general(x, gw_ref[...], (((1,), (1,)), ((), ())),
                             preferred_element_type=jnp.float32)
    s = jax.nn.sigmoid(logits)
    sc = s + bias_ref[...]
    ii = lax.broadcasted_iota(jnp.int32, (BT_R, E), 1)
    m1 = jnp.max(sc, axis=1, keepdims=True)
    i1 = jnp.min(jnp.where(sc >= m1, ii, E), axis=1, keepdims=True)
    sc2 = jnp.where(ii == i1, -jnp.inf, sc)
    m2 = jnp.max(sc2, axis=1, keepdims=True)
    i2 = jnp.min(jnp.where(sc2 >= m2, ii, E), axis=1, keepdims=True)
    s1 = jnp.sum(jnp.where(ii == i1, s, 0.0), axis=1, keepdims=True)
    s2 = jnp.sum(jnp.where(ii == i2, s, 0.0), axis=1, keepdims=True)
    denom = s1 + s2
    meta_ref[...] = jnp.concatenate(
        [i1.astype(jnp.float32), i2.astype(jnp.float32),
         s1 / denom, s2 / denom,
         jnp.zeros((BT_R, 4), jnp.float32)], axis=1)


def _router(x, gw, bias):
    return pl.pallas_call(
        _router_body,
        grid=(T // BT_R,),
        in_specs=[
            pl.BlockSpec((BT_R, H), lambda t: (t, 0)),
            pl.BlockSpec((E, H), lambda t: (0, 0)),
            pl.BlockSpec((1, E), lambda t: (0, 0)),
        ],
        out_specs=pl.BlockSpec((BT_R, 8), lambda t: (t, 0)),
        out_shape=jax.ShapeDtypeStruct((T, 8), jnp.float32),
        compiler_params=pltpu.CompilerParams(
            dimension_semantics=("arbitrary",)),
    )(x, gw, bias)


def _shared_body(x_ref, sgu_ref, sd_ref, out_ref):
    xb = x_ref[...].astype(jnp.bfloat16)
    gu = jnp.dot(xb, sgu_ref[...], preferred_element_type=jnp.float32)
    h = (jax.nn.silu(gu[:, :I]) * gu[:, I:]).astype(jnp.bfloat16)
    out_ref[...] = jnp.dot(h, sd_ref[...], preferred_element_type=jnp.float32)


def _shared(x, sgu16, sd16):
    return pl.pallas_call(
        _shared_body,
        grid=(T // BT_R,),
        in_specs=[
            pl.BlockSpec((BT_R, H), lambda t: (t, 0)),
            pl.BlockSpec((H, 2 * I), lambda t: (0, 0)),
            pl.BlockSpec((I, H), lambda t: (0, 0)),
        ],
        out_specs=pl.BlockSpec((BT_R, H), lambda t: (t, 0)),
        out_shape=jax.ShapeDtypeStruct((T, H), jnp.float32),
        compiler_params=pltpu.CompilerParams(
            dimension_semantics=("arbitrary",)),
    )(x, sgu16, sd16)


_PAIRS_W = TK // NW_PLAN
_ZERO_W = P // NW_PLAN
_META_W = 8 * (T // NW_PLAN)


def _plan_body(meta_hbm, tok_hbm, wts_hbm, blk_hbm, pos_hbm, hstage_hbm,
               mv, slot_v, tok_v, wt_v, zvi, zvf, hist_v, allh_v, blk_v,
               sem):
    wid = lax.axis_index("s")
    lane = lax.broadcasted_iota(jnp.int32, (NLANE,), 0)
    zeros16i = jnp.zeros((NLANE,), jnp.int32)

    for i in range(_ZERO_W // NLANE):
        zvi[pl.ds(NLANE * i, NLANE)] = zeros16i
        zvf[pl.ds(NLANE * i, NLANE)] = jnp.zeros((NLANE,), jnp.float32)
    pltpu.sync_copy(zvi, tok_hbm.at[pl.ds(wid * _ZERO_W, _ZERO_W)])
    pltpu.sync_copy(zvf, wts_hbm.at[pl.ds(wid * _ZERO_W, _ZERO_W)])

    pltpu.sync_copy(meta_hbm.at[pl.ds(wid * _META_W, _META_W)], mv)
    h = zeros16i
    for j in range(_PAIRS_W // NLANE):
        pi = j * NLANE + lane
        idx_e = ((pi >> 1) << 3) | (pi & 1)
        v = plsc.load_gather(mv, [idx_e]).astype(jnp.int32)
        for e in range(E):
            cnt = jnp.sum(jnp.where(v == e, 1, 0))
            h = h + jnp.where(lane == e, cnt, 0)
    hist_v[...] = h
    pltpu.sync_copy(hist_v, hstage_hbm.at[pl.ds(wid * NLANE, NLANE)])
    plsc.subcore_barrier()

    pltpu.sync_copy(hstage_hbm, allh_v)
    c_tot = zeros16i
    prior = zeros16i
    for w in range(NW_PLAN):
        row = allh_v[pl.ds(w * NLANE, NLANE)]
        c_tot = c_tot + row
        prior = prior + jnp.where(jnp.broadcast_to(w < wid, (NLANE,)), row, 0)
    r = ((c_tot + (BT - 1)) >> 8) << 8
    start = plsc.cumsum(r) - r
    b_run = start + prior

    for j in range(_PAIRS_W // NLANE):
        pi = j * NLANE + lane
        idx_e = ((pi >> 1) << 3) | (pi & 1)
        v = plsc.load_gather(mv, [idx_e]).astype(jnp.int32)
        w_f = plsc.load_gather(mv, [idx_e + 2])
        slot = zeros16i
        for e in range(E):
            ind = jnp.where(v == e, 1, 0)
            prefix = plsc.cumsum(ind)
            cnt = jnp.sum(ind)
            base_e = jnp.sum(jnp.where(lane == e, b_run, 0))
            slot = jnp.where(v == e, base_e + prefix - 1, slot)
            b_run = b_run + jnp.where(lane == e, cnt, 0)
        sl = pl.ds(j * NLANE, NLANE)
        slot_v[sl] = slot
        tok_v[sl] = wid * (_PAIRS_W // 2) + (pi >> 1)
        wt_v[sl] = w_f
    pltpu.async_copy(tok_v, tok_hbm.at[slot_v], sem).wait()
    pltpu.async_copy(wt_v, wts_hbm.at[slot_v], sem).wait()
    pltpu.sync_copy(slot_v, pos_hbm.at[pl.ds(wid * _PAIRS_W, _PAIRS_W)])

    @pl.when(wid == 0)
    def _():
        nb = (c_tot + (BT - 1)) >> 8
        sb = start >> 8
        for j in range(NBP // NLANE):
            bi = j * NLANE + lane
            acc = zeros16i
            for e in range(1, E):
                sb_e = jnp.sum(jnp.where(lane == e, sb, 0))
                nb_e = jnp.sum(jnp.where(lane == e, nb, 0))
                m = (bi >= sb_e) & (bi < sb_e + nb_e)
                acc = acc + jnp.where(m, e, 0)
            blk_v[pl.ds(j * NLANE, NLANE)] = acc
        pltpu.sync_copy(blk_v, blk_hbm)


def _plan(meta_flat):
    f = pl.kernel(
        _plan_body,
        out_type=[
            jax.ShapeDtypeStruct((P,), jnp.int32),
            jax.ShapeDtypeStruct((P,), jnp.float32),
            jax.ShapeDtypeStruct((NBP,), jnp.int32),
            jax.ShapeDtypeStruct((TK,), jnp.int32),
            jax.ShapeDtypeStruct((NW_PLAN * NLANE,), jnp.int32),
        ],
        mesh=plsc.VectorSubcoreMesh(core_axis_name="c", subcore_axis_name="s",
                                    num_cores=1),
        scratch_types=[
            pltpu.VMEM((_META_W,), jnp.float32),
            pltpu.VMEM((_PAIRS_W,), jnp.int32),
            pltpu.VMEM((_PAIRS_W,), jnp.int32),
            pltpu.VMEM((_PAIRS_W,), jnp.float32),
            pltpu.VMEM((_ZERO_W,), jnp.int32),
            pltpu.VMEM((_ZERO_W,), jnp.float32),
            pltpu.VMEM((NLANE,), jnp.int32),
            pltpu.VMEM((NW_PLAN * NLANE,), jnp.int32),
            pltpu.VMEM((NBP,), jnp.int32),
            pltpu.SemaphoreType.DMA,
        ],
        compiler_params=pltpu.CompilerParams(needs_layout_passes=False),
    )
    return f(meta_flat)


_ROWS_W = P // NW
_GCH = 40
_GNC = _ROWS_W // _GCH


def _gather_body(tok_hbm, x_hbm, xg_hbm, idx_v, row0, row1,
                 gs0, gs1, ws0, ws1):
    wid = lax.axis_index("s") * 2 + lax.axis_index("c")
    base = wid * _ROWS_W
    rows = (row0, row1)
    gsem = (gs0, gs1)
    wsem = (ws0, ws1)
    pltpu.sync_copy(tok_hbm.at[pl.ds(base, _ROWS_W)], idx_v)
    gq = [None, None]
    wq = [None, None]
    for c in range(_GNC):
        b = c % 2
        if wq[b] is not None:
            wq[b].wait()
        gq[b] = pltpu.async_copy(
            x_hbm.at[idx_v.at[pl.ds(c * _GCH, _GCH)]], rows[b], gsem[b])
        if c >= 1:
            pb = (c - 1) % 2
            gq[pb].wait()
            wq[pb] = pltpu.async_copy(
                rows[pb], xg_hbm.at[pl.ds(base + (c - 1) * _GCH, _GCH)],
                wsem[pb])
    lb = (_GNC - 1) % 2
    gq[lb].wait()
    pltpu.sync_copy(rows[lb], xg_hbm.at[pl.ds(base + (_GNC - 1) * _GCH, _GCH)])
    if wq[1 - lb] is not None:
        wq[1 - lb].wait()


def _gather(tok, x):
    f = pl.kernel(
        _gather_body,
        out_type=[jax.ShapeDtypeStruct((P, H), jnp.float32)],
        mesh=plsc.VectorSubcoreMesh(core_axis_name="c", subcore_axis_name="s"),
        scratch_types=[
            pltpu.VMEM((_ROWS_W,), jnp.int32),
            pltpu.VMEM((_GCH, H), jnp.float32),
            pltpu.VMEM((_GCH, H), jnp.float32),
            pltpu.SemaphoreType.DMA,
            pltpu.SemaphoreType.DMA,
            pltpu.SemaphoreType.DMA,
            pltpu.SemaphoreType.DMA,
        ],
        compiler_params=pltpu.CompilerParams(needs_layout_passes=False),
    )
    return f(tok, x)


def _ffn_body(be_ref, xg_ref, wgu_ref, wd_ref, wt_ref, out_ref):
    xb = xg_ref[...].astype(jnp.bfloat16)
    gu = jnp.dot(xb, wgu_ref[0], preferred_element_type=jnp.float32)
    h = (jax.nn.silu(gu[:, :I]) * gu[:, I:]).astype(jnp.bfloat16)
    oe = jnp.dot(h, wd_ref[0], preferred_element_type=jnp.float32)
    out_ref[...] = oe * wt_ref[0, 0, :][:, None]


def _ffn(blk, xg, wgu16, wd16, wt3):
    grid_spec = pltpu.PrefetchScalarGridSpec(
        num_scalar_prefetch=1,
        grid=(NB,),
        in_specs=[
            pl.BlockSpec((BT, H), lambda b, be: (b, 0)),
            pl.BlockSpec((1, H, 2 * I), lambda b, be: (be[b], 0, 0)),
            pl.BlockSpec((1, I, H), lambda b, be: (be[b], 0, 0)),
            pl.BlockSpec((1, 1, BT), lambda b, be: (b, 0, 0)),
        ],
        out_specs=pl.BlockSpec((BT, H), lambda b, be: (b, 0)),
    )
    return pl.pallas_call(
        _ffn_body,
        grid_spec=grid_spec,
        out_shape=jax.ShapeDtypeStruct((P, H), jnp.float32),
        compiler_params=pltpu.CompilerParams(
            dimension_semantics=("arbitrary",)),
    )(blk, xg, wgu16, wd16, wt3)


_TOK_W = T // NW
_CCH = 16
_CNC = _TOK_W // _CCH


def _combine_body(pos_hbm, routed_hbm, sh_hbm, out_hbm, idx_v,
                  g0, g1, a0, a1, gs0, gs1, ss0, ss1, ws0, ws1):
    wid = lax.axis_index("s") * 2 + lax.axis_index("c")
    tb0 = wid * _TOK_W
    g_b = (g0, g1)
    a_b = (a0, a1)
    gsem = (gs0, gs1)
    ssem = (ss0, ss1)
    wsem = (ws0, ws1)
    pltpu.sync_copy(pos_hbm.at[pl.ds(2 * tb0, 2 * _TOK_W)], idx_v)

    def compute(c, b):
        g_v = g_b[b]
        acc_v = a_b[b]

        def body(i, carry):
            for jj in range(H // NLANE):
                sl = pl.ds(NLANE * jj, NLANE)
                plsc.addupdate(acc_v.at[i, sl], g_v[2 * i, sl] + g_v[2 * i + 1, sl])
            return carry

        lax.fori_loop(0, _CCH, body, 0)

    gq = [None, None]
    sq = [None, None]
    wq = [None, None]
    for c in range(_CNC):
        b = c % 2
        if wq[b] is not None:
            wq[b].wait()
        tb = tb0 + c * _CCH
        gq[b] = pltpu.async_copy(
            routed_hbm.at[idx_v.at[pl.ds(2 * c * _CCH, 2 * _CCH)]], g_b[b],
            gsem[b])
        sq[b] = pltpu.async_copy(sh_hbm.at[pl.ds(tb, _CCH)], a_b[b], ssem[b])
        if c >= 1:
            pb = (c - 1) % 2
            gq[pb].wait()
            sq[pb].wait()
            compute(c - 1, pb)
            wq[pb] = pltpu.async_copy(
                a_b[pb], out_hbm.at[pl.ds(tb0 + (c - 1) * _CCH, _CCH)],
                wsem[pb])
    lb = (_CNC - 1) % 2
    gq[lb].wait()
    sq[lb].wait()
    compute(_CNC - 1, lb)
    pltpu.sync_copy(a_b[lb], out_hbm.at[pl.ds(tb0 + (_CNC - 1) * _CCH, _CCH)])
    if wq[1 - lb] is not None:
        wq[1 - lb].wait()


def _combine(pos, routed, shared_out):
    f = pl.kernel(
        _combine_body,
        out_type=[jax.ShapeDtypeStruct((T, H), jnp.float32)],
        mesh=plsc.VectorSubcoreMesh(core_axis_name="c", subcore_axis_name="s"),
        scratch_types=[
            pltpu.VMEM((2 * _TOK_W,), jnp.int32),
            pltpu.VMEM((2 * _CCH, H), jnp.float32),
            pltpu.VMEM((2 * _CCH, H), jnp.float32),
            pltpu.VMEM((_CCH, H), jnp.float32),
            pltpu.VMEM((_CCH, H), jnp.float32),
            pltpu.SemaphoreType.DMA,
            pltpu.SemaphoreType.DMA,
            pltpu.SemaphoreType.DMA,
            pltpu.SemaphoreType.DMA,
            pltpu.SemaphoreType.DMA,
            pltpu.SemaphoreType.DMA,
        ],
        compiler_params=pltpu.CompilerParams(needs_layout_passes=False),
    )
    return f(pos, routed, shared_out)


def kernel(hidden_states, gate_weight, correction_bias, w_gate_up, w_down,
           shared_gate_up, shared_down):
    bf = jnp.bfloat16
    x = hidden_states
    meta = _router(x, gate_weight, correction_bias)
    shared_out = _shared(x, shared_gate_up.astype(bf), shared_down.astype(bf))
    tok, wt, blk, pos, _ = _plan(meta.reshape(-1))
    (xg,) = _gather(tok, x)
    routed = _ffn(blk, xg, w_gate_up.astype(bf), w_down.astype(bf),
                  wt.reshape(NB, 1, BT))
    (out,) = _combine(pos, routed, shared_out)
    return out


# speedup vs baseline: 1.1329x; 1.0689x over previous
import functools

import jax
import jax.numpy as jnp
from jax import lax
from jax.experimental import pallas as pl
from jax.experimental.pallas import tpu as pltpu
from jax.experimental.pallas import tpu_sc as plsc

T = 4096
H = 1024
I = 512
E = 8
TOP_K = 2

BT = 256
P = T * TOP_K + E * BT
NB = P // BT
NBP = 48
TK = T * TOP_K

NLANE = 16
NW_PLAN = 16
NW = 32

BT_R = 512


def _router_body(x_ref, gw_ref, bias_ref, meta_ref, xb_ref):
    x = x_ref[...]
    u = lax.bitcast_convert_type(x, jnp.int32) + 0x8000
    hi_a = u[:, :I] & -65536
    hi_b = u[:, I:] & -65536
    packed = hi_a | lax.shift_right_logical(hi_b, 16)
    xb_ref[...] = lax.bitcast_convert_type(packed, jnp.float32)
    logits = lax.dot_general(x, gw_ref[...], (((1,), (1,)), ((), ())),
                             preferred_element_type=jnp.float32)
    s = jax.nn.sigmoid(logits)
    sc = s + bias_ref[...]
    ii = lax.broadcasted_iota(jnp.int32, (BT_R, E), 1)
    m1 = jnp.max(sc, axis=1, keepdims=True)
    i1 = jnp.min(jnp.where(sc >= m1, ii, E), axis=1, keepdims=True)
    sc2 = jnp.where(ii == i1, -jnp.inf, sc)
    m2 = jnp.max(sc2, axis=1, keepdims=True)
    i2 = jnp.min(jnp.where(sc2 >= m2, ii, E), axis=1, keepdims=True)
    s1 = jnp.sum(jnp.where(ii == i1, s, 0.0), axis=1, keepdims=True)
    s2 = jnp.sum(jnp.where(ii == i2, s, 0.0), axis=1, keepdims=True)
    denom = s1 + s2
    meta_ref[...] = jnp.concatenate(
        [i1.astype(jnp.float32), i2.astype(jnp.float32),
         s1 / denom, s2 / denom,
         jnp.zeros((BT_R, 4), jnp.float32)], axis=1)


def _router(x, gw, bias):
    return pl.pallas_call(
        _router_body,
        grid=(T // BT_R,),
        in_specs=[
            pl.BlockSpec((BT_R, H), lambda t: (t, 0)),
            pl.BlockSpec((E, H), lambda t: (0, 0)),
            pl.BlockSpec((1, E), lambda t: (0, 0)),
        ],
        out_specs=[pl.BlockSpec((BT_R, 8), lambda t: (t, 0)),
                   pl.BlockSpec((BT_R, I), lambda t: (t, 0))],
        out_shape=[jax.ShapeDtypeStruct((T, 8), jnp.float32),
                   jax.ShapeDtypeStruct((T, I), jnp.float32)],
        compiler_params=pltpu.CompilerParams(
            dimension_semantics=("arbitrary",)),
    )(x, gw, bias)


def _shared_body(x_ref, sgu_ref, sd_ref, out_ref):
    xb = x_ref[...].astype(jnp.bfloat16)
    gu = jnp.dot(xb, sgu_ref[...], preferred_element_type=jnp.float32)
    h = (jax.nn.silu(gu[:, :I]) * gu[:, I:]).astype(jnp.bfloat16)
    out_ref[...] = jnp.dot(h, sd_ref[...], preferred_element_type=jnp.float32)


def _shared(x, sgu16, sd16):
    return pl.pallas_call(
        _shared_body,
        grid=(T // BT_R,),
        in_specs=[
            pl.BlockSpec((BT_R, H), lambda t: (t, 0)),
            pl.BlockSpec((H, 2 * I), lambda t: (0, 0)),
            pl.BlockSpec((I, H), lambda t: (0, 0)),
        ],
        out_specs=pl.BlockSpec((BT_R, H), lambda t: (t, 0)),
        out_shape=jax.ShapeDtypeStruct((T, H), jnp.float32),
        compiler_params=pltpu.CompilerParams(
            dimension_semantics=("arbitrary",)),
    )(x, sgu16, sd16)


_PAIRS_W = TK // NW_PLAN
_ZERO_W = P // NW_PLAN
_META_W = 8 * (T // NW_PLAN)


def _plan_body(meta_hbm, tok_hbm, wts_hbm, blk_hbm, pos_hbm, hstage_hbm,
               mv, slot_v, tok_v, wt_v, zvi, zvf, hist_v, allh_v, blk_v,
               sem):
    wid = lax.axis_index("s")
    lane = lax.broadcasted_iota(jnp.int32, (NLANE,), 0)
    zeros16i = jnp.zeros((NLANE,), jnp.int32)

    for i in range(_ZERO_W // NLANE):
        zvi[pl.ds(NLANE * i, NLANE)] = zeros16i
        zvf[pl.ds(NLANE * i, NLANE)] = jnp.zeros((NLANE,), jnp.float32)
    pltpu.sync_copy(zvi, tok_hbm.at[pl.ds(wid * _ZERO_W, _ZERO_W)])
    pltpu.sync_copy(zvf, wts_hbm.at[pl.ds(wid * _ZERO_W, _ZERO_W)])

    pltpu.sync_copy(meta_hbm.at[pl.ds(wid * _META_W, _META_W)], mv)
    h = zeros16i
    for j in range(_PAIRS_W // NLANE):
        pi = j * NLANE + lane
        idx_e = ((pi >> 1) << 3) | (pi & 1)
        v = plsc.load_gather(mv, [idx_e]).astype(jnp.int32)
        for e in range(E):
            cnt = jnp.sum(jnp.where(v == e, 1, 0))
            h = h + jnp.where(lane == e, cnt, 0)
    hist_v[...] = h
    pltpu.sync_copy(hist_v, hstage_hbm.at[pl.ds(wid * NLANE, NLANE)])
    plsc.subcore_barrier()

    pltpu.sync_copy(hstage_hbm, allh_v)
    c_tot = zeros16i
    prior = zeros16i
    for w in range(NW_PLAN):
        row = allh_v[pl.ds(w * NLANE, NLANE)]
        c_tot = c_tot + row
        prior = prior + jnp.where(jnp.broadcast_to(w < wid, (NLANE,)), row, 0)
    r = ((c_tot + (BT - 1)) >> 8) << 8
    start = plsc.cumsum(r) - r
    b_run = start + prior

    for j in range(_PAIRS_W // NLANE):
        pi = j * NLANE + lane
        idx_e = ((pi >> 1) << 3) | (pi & 1)
        v = plsc.load_gather(mv, [idx_e]).astype(jnp.int32)
        w_f = plsc.load_gather(mv, [idx_e + 2])
        slot = zeros16i
        for e in range(E):
            ind = jnp.where(v == e, 1, 0)
            prefix = plsc.cumsum(ind)
            cnt = jnp.sum(ind)
            base_e = jnp.sum(jnp.where(lane == e, b_run, 0))
            slot = jnp.where(v == e, base_e + prefix - 1, slot)
            b_run = b_run + jnp.where(lane == e, cnt, 0)
        sl = pl.ds(j * NLANE, NLANE)
        slot_v[sl] = slot
        tok_v[sl] = wid * (_PAIRS_W // 2) + (pi >> 1)
        wt_v[sl] = w_f
    pltpu.async_copy(tok_v, tok_hbm.at[slot_v], sem).wait()
    pltpu.async_copy(wt_v, wts_hbm.at[slot_v], sem).wait()
    pltpu.sync_copy(slot_v, pos_hbm.at[pl.ds(wid * _PAIRS_W, _PAIRS_W)])

    @pl.when(wid == 0)
    def _():
        nb = (c_tot + (BT - 1)) >> 8
        sb = start >> 8
        for j in range(NBP // NLANE):
            bi = j * NLANE + lane
            acc = zeros16i
            for e in range(1, E):
                sb_e = jnp.sum(jnp.where(lane == e, sb, 0))
                nb_e = jnp.sum(jnp.where(lane == e, nb, 0))
                m = (bi >= sb_e) & (bi < sb_e + nb_e)
                acc = acc + jnp.where(m, e, 0)
            blk_v[pl.ds(j * NLANE, NLANE)] = acc
        pltpu.sync_copy(blk_v, blk_hbm)


def _plan(meta_flat):
    f = pl.kernel(
        _plan_body,
        out_type=[
            jax.ShapeDtypeStruct((P,), jnp.int32),
            jax.ShapeDtypeStruct((P,), jnp.float32),
            jax.ShapeDtypeStruct((NBP,), jnp.int32),
            jax.ShapeDtypeStruct((TK,), jnp.int32),
            jax.ShapeDtypeStruct((NW_PLAN * NLANE,), jnp.int32),
        ],
        mesh=plsc.VectorSubcoreMesh(core_axis_name="c", subcore_axis_name="s",
                                    num_cores=1),
        scratch_types=[
            pltpu.VMEM((_META_W,), jnp.float32),
            pltpu.VMEM((_PAIRS_W,), jnp.int32),
            pltpu.VMEM((_PAIRS_W,), jnp.int32),
            pltpu.VMEM((_PAIRS_W,), jnp.float32),
            pltpu.VMEM((_ZERO_W,), jnp.int32),
            pltpu.VMEM((_ZERO_W,), jnp.float32),
            pltpu.VMEM((NLANE,), jnp.int32),
            pltpu.VMEM((NW_PLAN * NLANE,), jnp.int32),
            pltpu.VMEM((NBP,), jnp.int32),
            pltpu.SemaphoreType.DMA,
        ],
        compiler_params=pltpu.CompilerParams(needs_layout_passes=False),
    )
    return f(meta_flat)


_ROWS_W = P // NW
_GCH = 40
_GNC = _ROWS_W // _GCH


def _gather_body(tok_hbm, x_hbm, xg_hbm, idx_v, row0, row1,
                 gs0, gs1, ws0, ws1):
    wid = lax.axis_index("s") * 2 + lax.axis_index("c")
    base = wid * _ROWS_W
    rows = (row0, row1)
    gsem = (gs0, gs1)
    wsem = (ws0, ws1)
    pltpu.sync_copy(tok_hbm.at[pl.ds(base, _ROWS_W)], idx_v)
    gq = [None, None]
    wq = [None, None]
    for c in range(_GNC):
        b = c % 2
        if wq[b] is not None:
            wq[b].wait()
        gq[b] = pltpu.async_copy(
            x_hbm.at[idx_v.at[pl.ds(c * _GCH, _GCH)]], rows[b], gsem[b])
        if c >= 1:
            pb = (c - 1) % 2
            gq[pb].wait()
            wq[pb] = pltpu.async_copy(
                rows[pb], xg_hbm.at[pl.ds(base + (c - 1) * _GCH, _GCH)],
                wsem[pb])
    lb = (_GNC - 1) % 2
    gq[lb].wait()
    pltpu.sync_copy(rows[lb], xg_hbm.at[pl.ds(base + (_GNC - 1) * _GCH, _GCH)])
    if wq[1 - lb] is not None:
        wq[1 - lb].wait()


def _gather(tok, x):
    f = pl.kernel(
        _gather_body,
        out_type=[jax.ShapeDtypeStruct((P, I), jnp.float32)],
        mesh=plsc.VectorSubcoreMesh(core_axis_name="c", subcore_axis_name="s"),
        scratch_types=[
            pltpu.VMEM((_ROWS_W,), jnp.int32),
            pltpu.VMEM((_GCH, I), jnp.float32),
            pltpu.VMEM((_GCH, I), jnp.float32),
            pltpu.SemaphoreType.DMA,
            pltpu.SemaphoreType.DMA,
            pltpu.SemaphoreType.DMA,
            pltpu.SemaphoreType.DMA,
        ],
        compiler_params=pltpu.CompilerParams(needs_layout_passes=False),
    )
    return f(tok, x)


def _ffn_body(be_ref, xg_ref, wgu_ref, wd_ref, wt_ref, out_ref):
    u = lax.bitcast_convert_type(xg_ref[...], jnp.int32)
    a = lax.bitcast_convert_type(u & -65536, jnp.float32)
    b = lax.bitcast_convert_type(lax.shift_left(u, 16), jnp.float32)
    xb = jnp.concatenate([a, b], axis=1).astype(jnp.bfloat16)
    gu = jnp.dot(xb, wgu_ref[0], preferred_element_type=jnp.float32)
    h = (jax.nn.silu(gu[:, :I]) * gu[:, I:]).astype(jnp.bfloat16)
    oe = jnp.dot(h, wd_ref[0], preferred_element_type=jnp.float32)
    out_ref[...] = oe * wt_ref[0, 0, :][:, None]


def _ffn(blk, xg, wgu16, wd16, wt3):
    grid_spec = pltpu.PrefetchScalarGridSpec(
        num_scalar_prefetch=1,
        grid=(NB,),
        in_specs=[
            pl.BlockSpec((BT, I), lambda b, be: (b, 0)),
            pl.BlockSpec((1, H, 2 * I), lambda b, be: (be[b], 0, 0)),
            pl.BlockSpec((1, I, H), lambda b, be: (be[b], 0, 0)),
            pl.BlockSpec((1, 1, BT), lambda b, be: (b, 0, 0)),
        ],
        out_specs=pl.BlockSpec((BT, H), lambda b, be: (b, 0)),
    )
    return pl.pallas_call(
        _ffn_body,
        grid_spec=grid_spec,
        out_shape=jax.ShapeDtypeStruct((P, H), jnp.float32),
        compiler_params=pltpu.CompilerParams(
            dimension_semantics=("arbitrary",)),
    )(blk, xg, wgu16, wd16, wt3)


_TOK_W = T // NW
_CCH = 16
_CNC = _TOK_W // _CCH


def _combine_body(pos_hbm, routed_hbm, sh_hbm, out_hbm, idx_v,
                  g0, g1, a0, a1, gs0, gs1, ss0, ss1, ws0, ws1):
    wid = lax.axis_index("s") * 2 + lax.axis_index("c")
    tb0 = wid * _TOK_W
    g_b = (g0, g1)
    a_b = (a0, a1)
    gsem = (gs0, gs1)
    ssem = (ss0, ss1)
    wsem = (ws0, ws1)
    pltpu.sync_copy(pos_hbm.at[pl.ds(2 * tb0, 2 * _TOK_W)], idx_v)

    def compute(c, b):
        g_v = g_b[b]
        acc_v = a_b[b]

        def body(i, carry):
            for jj in range(H // NLANE):
                sl = pl.ds(NLANE * jj, NLANE)
                plsc.addupdate(acc_v.at[i, sl], g_v[2 * i, sl] + g_v[2 * i + 1, sl])
            return carry

        lax.fori_loop(0, _CCH, body, 0)

    gq = [None, None]
    sq = [None, None]
    wq = [None, None]
    for c in range(_CNC):
        b = c % 2
        if wq[b] is not None:
            wq[b].wait()
        tb = tb0 + c * _CCH
        gq[b] = pltpu.async_copy(
            routed_hbm.at[idx_v.at[pl.ds(2 * c * _CCH, 2 * _CCH)]], g_b[b],
            gsem[b])
        sq[b] = pltpu.async_copy(sh_hbm.at[pl.ds(tb, _CCH)], a_b[b], ssem[b])
        if c >= 1:
            pb = (c - 1) % 2
            gq[pb].wait()
            sq[pb].wait()
            compute(c - 1, pb)
            wq[pb] = pltpu.async_copy(
                a_b[pb], out_hbm.at[pl.ds(tb0 + (c - 1) * _CCH, _CCH)],
                wsem[pb])
    lb = (_CNC - 1) % 2
    gq[lb].wait()
    sq[lb].wait()
    compute(_CNC - 1, lb)
    pltpu.sync_copy(a_b[lb], out_hbm.at[pl.ds(tb0 + (_CNC - 1) * _CCH, _CCH)])
    if wq[1 - lb] is not None:
        wq[1 - lb].wait()


def _combine(pos, routed, shared_out):
    f = pl.kernel(
        _combine_body,
        out_type=[jax.ShapeDtypeStruct((T, H), jnp.float32)],
        mesh=plsc.VectorSubcoreMesh(core_axis_name="c", subcore_axis_name="s"),
        scratch_types=[
            pltpu.VMEM((2 * _TOK_W,), jnp.int32),
            pltpu.VMEM((2 * _CCH, H), jnp.float32),
            pltpu.VMEM((2 * _CCH, H), jnp.float32),
            pltpu.VMEM((_CCH, H), jnp.float32),
            pltpu.VMEM((_CCH, H), jnp.float32),
            pltpu.SemaphoreType.DMA,
            pltpu.SemaphoreType.DMA,
            pltpu.SemaphoreType.DMA,
            pltpu.SemaphoreType.DMA,
            pltpu.SemaphoreType.DMA,
            pltpu.SemaphoreType.DMA,
        ],
        compiler_params=pltpu.CompilerParams(needs_layout_passes=False),
    )
    return f(pos, routed, shared_out)


def kernel(hidden_states, gate_weight, correction_bias, w_gate_up, w_down,
           shared_gate_up, shared_down):
    bf = jnp.bfloat16
    x = hidden_states
    meta, xb16 = _router(x, gate_weight, correction_bias)
    shared_out = _shared(x, shared_gate_up.astype(bf), shared_down.astype(bf))
    tok, wt, blk, pos, _ = _plan(meta.reshape(-1))
    (xg,) = _gather(tok, xb16)
    routed = _ffn(blk, xg, w_gate_up.astype(bf), w_down.astype(bf),
                  wt.reshape(NB, 1, BT))
    (out,) = _combine(pos, routed, shared_out)
    return out


# speedup vs baseline: 2.6077x; 2.3017x over previous
import functools

import jax
import jax.numpy as jnp
from jax import lax
from jax.experimental import pallas as pl
from jax.experimental.pallas import tpu as pltpu

T = 4096
H = 1024
I = 512
E = 8
TOP_K = 2

BT = 512


def _dense_body(x_ref, gw_ref, bias_ref, wgu_ref, wd_ref, sgu_ref, sd_ref, out_ref):
    x = x_ref[...]

    logits = lax.dot_general(x, gw_ref[...], (((1,), (1,)), ((), ())),
                             preferred_element_type=jnp.float32)
    s = jax.nn.sigmoid(logits)
    sc = s + bias_ref[...]
    ii = lax.broadcasted_iota(jnp.int32, (BT, E), 1)
    m1 = jnp.max(sc, axis=1, keepdims=True)
    i1 = jnp.min(jnp.where(sc >= m1, ii, E), axis=1, keepdims=True)
    sc2 = jnp.where(ii == i1, -jnp.inf, sc)
    m2 = jnp.max(sc2, axis=1, keepdims=True)
    i2 = jnp.min(jnp.where(sc2 >= m2, ii, E), axis=1, keepdims=True)
    s1 = jnp.sum(jnp.where(ii == i1, s, 0.0), axis=1, keepdims=True)
    s2 = jnp.sum(jnp.where(ii == i2, s, 0.0), axis=1, keepdims=True)
    denom = s1 + s2
    w1 = s1 / denom
    w2 = s2 / denom

    xb = x.astype(jnp.bfloat16)

    sgu = jnp.dot(xb, sgu_ref[...], preferred_element_type=jnp.float32)
    sh = (jax.nn.silu(sgu[:, :I]) * sgu[:, I:]).astype(jnp.bfloat16)
    acc = jnp.dot(sh, sd_ref[...], preferred_element_type=jnp.float32)

    for e in range(E):
        w_e = jnp.where(i1 == e, w1, 0.0) + jnp.where(i2 == e, w2, 0.0)
        gu = jnp.dot(xb, wgu_ref[e], preferred_element_type=jnp.float32)
        h = (jax.nn.silu(gu[:, :I]) * gu[:, I:]).astype(jnp.bfloat16)
        oe = jnp.dot(h, wd_ref[e], preferred_element_type=jnp.float32)
        acc = acc + w_e * oe

    out_ref[...] = acc


@functools.partial(jax.jit, static_argnames=("interpret",))
def _moe_dense(x, gate_weight, bias, w_gate_up, w_down, sgu, sd, interpret=False):
    grid = (T // BT,)
    return pl.pallas_call(
        _dense_body,
        grid=grid,
        in_specs=[
            pl.BlockSpec((BT, H), lambda t: (t, 0)),
            pl.BlockSpec((E, H), lambda t: (0, 0)),
            pl.BlockSpec((1, E), lambda t: (0, 0)),
            pl.BlockSpec((E, H, 2 * I), lambda t: (0, 0, 0)),
            pl.BlockSpec((E, I, H), lambda t: (0, 0, 0)),
            pl.BlockSpec((H, 2 * I), lambda t: (0, 0)),
            pl.BlockSpec((I, H), lambda t: (0, 0)),
        ],
        out_specs=pl.BlockSpec((BT, H), lambda t: (t, 0)),
        out_shape=jax.ShapeDtypeStruct((T, H), jnp.float32),
        compiler_params=pltpu.CompilerParams(
            dimension_semantics=("arbitrary",),
        ),
        interpret=interpret,
    )(x, gate_weight, bias, w_gate_up, w_down, sgu, sd)


def kernel(hidden_states, gate_weight, correction_bias, w_gate_up, w_down,
           shared_gate_up, shared_down):
    bf = jnp.bfloat16
    return _moe_dense(hidden_states, gate_weight, correction_bias,
                      w_gate_up.astype(bf), w_down.astype(bf),
                      shared_gate_up.astype(bf), shared_down.astype(bf))


# speedup vs baseline: 2.6171x; 1.0036x over previous
import functools

import jax
import jax.numpy as jnp
from jax import lax
from jax.experimental import pallas as pl
from jax.experimental.pallas import tpu as pltpu

T = 4096
H = 1024
I = 512
E = 8
TOP_K = 2

BT = 1024


def _dense_body(x_ref, gw_ref, bias_ref, wgu_ref, wd_ref, sgu_ref, sd_ref, out_ref):
    x = x_ref[...]

    logits = lax.dot_general(x, gw_ref[...], (((1,), (1,)), ((), ())),
                             preferred_element_type=jnp.float32)
    s = jax.nn.sigmoid(logits)
    sc = s + bias_ref[...]
    ii = lax.broadcasted_iota(jnp.int32, (BT, E), 1)
    m1 = jnp.max(sc, axis=1, keepdims=True)
    i1 = jnp.min(jnp.where(sc >= m1, ii, E), axis=1, keepdims=True)
    sc2 = jnp.where(ii == i1, -jnp.inf, sc)
    m2 = jnp.max(sc2, axis=1, keepdims=True)
    i2 = jnp.min(jnp.where(sc2 >= m2, ii, E), axis=1, keepdims=True)
    s1 = jnp.sum(jnp.where(ii == i1, s, 0.0), axis=1, keepdims=True)
    s2 = jnp.sum(jnp.where(ii == i2, s, 0.0), axis=1, keepdims=True)
    denom = s1 + s2
    w1 = s1 / denom
    w2 = s2 / denom

    xb = x.astype(jnp.bfloat16)

    sgu = jnp.dot(xb, sgu_ref[...], preferred_element_type=jnp.float32)
    sh = (jax.nn.silu(sgu[:, :I]) * sgu[:, I:]).astype(jnp.bfloat16)
    acc = jnp.dot(sh, sd_ref[...], preferred_element_type=jnp.float32)

    for e in range(E):
        w_e = jnp.where(i1 == e, w1, 0.0) + jnp.where(i2 == e, w2, 0.0)
        gu = jnp.dot(xb, wgu_ref[e], preferred_element_type=jnp.float32)
        h = (jax.nn.silu(gu[:, :I]) * gu[:, I:]).astype(jnp.bfloat16)
        oe = jnp.dot(h, wd_ref[e], preferred_element_type=jnp.float32)
        acc = acc + w_e * oe

    out_ref[...] = acc


@functools.partial(jax.jit, static_argnames=("interpret",))
def _moe_dense(x, gate_weight, bias, w_gate_up, w_down, sgu, sd, interpret=False):
    grid = (T // BT,)
    return pl.pallas_call(
        _dense_body,
        grid=grid,
        in_specs=[
            pl.BlockSpec((BT, H), lambda t: (t, 0)),
            pl.BlockSpec((E, H), lambda t: (0, 0)),
            pl.BlockSpec((1, E), lambda t: (0, 0)),
            pl.BlockSpec((E, H, 2 * I), lambda t: (0, 0, 0)),
            pl.BlockSpec((E, I, H), lambda t: (0, 0, 0)),
            pl.BlockSpec((H, 2 * I), lambda t: (0, 0)),
            pl.BlockSpec((I, H), lambda t: (0, 0)),
        ],
        out_specs=pl.BlockSpec((BT, H), lambda t: (t, 0)),
        out_shape=jax.ShapeDtypeStruct((T, H), jnp.float32),
        compiler_params=pltpu.CompilerParams(
            dimension_semantics=("arbitrary",),
        ),
        interpret=interpret,
    )(x, gate_weight, bias, w_gate_up, w_down, sgu, sd)


def kernel(hidden_states, gate_weight, correction_bias, w_gate_up, w_down,
           shared_gate_up, shared_down):
    bf = jnp.bfloat16
    return _moe_dense(hidden_states, gate_weight, correction_bias,
                      w_gate_up.astype(bf), w_down.astype(bf),
                      shared_gate_up.astype(bf), shared_down.astype(bf))
